# baseline (speedup 1.0000x reference)
import functools

import jax
import jax.numpy as jnp
from jax import lax
from jax.experimental import pallas as pl
from jax.experimental.pallas import tpu as pltpu
from jax.experimental.pallas import tpu_sc as plsc

HIDDEN = 1024
SEQ = 2048
NTOK = 4 * SEQ
NC, NS = 2, 16
NW = NC * NS
BPW = NTOK // NW
C = 32
NCHUNK = BPW // C
LANES = 16
VPR = HIDDEN // LANES

_mesh = plsc.VectorSubcoreMesh(core_axis_name="c", subcore_axis_name="s")


@functools.partial(
    pl.kernel,
    out_type=jax.ShapeDtypeStruct((NTOK, HIDDEN), jnp.float32),
    mesh=_mesh,
    scratch_types=[
        pltpu.VMEM((BPW,), jnp.int32),
        pltpu.VMEM((C, HIDDEN), jnp.float32),
        pltpu.VMEM((C, HIDDEN), jnp.float32),
        pltpu.SemaphoreType.DMA,
    ],
)
def _embed(ids_hbm, wte_hbm, wpe_hbm, out_hbm, idx_v, rows_v, wpe_v, sem):
    wid = lax.axis_index("s") * NC + lax.axis_index("c")
    base = wid * BPW
    pos_base = lax.rem(base, SEQ)
    pltpu.sync_copy(ids_hbm.at[pl.ds(base, BPW)], idx_v)

    @pl.loop(0, NCHUNK)
    def _chunk(c):
        off = c * C
        gather = pltpu.async_copy(
            wte_hbm.at[idx_v.at[pl.ds(off, C)]], rows_v, sem)
        pltpu.sync_copy(wpe_hbm.at[pl.ds(pos_base + off, C)], wpe_v)
        gather.wait()

        @pl.loop(0, C)
        def _row(r):
            @pl.loop(0, VPR)
            def _seg(j):
                plsc.addupdate(rows_v.at[r, pl.ds(j * LANES, LANES)],
                               wpe_v[r, pl.ds(j * LANES, LANES)])

        pltpu.sync_copy(rows_v, out_hbm.at[pl.ds(base + off, C)])


def kernel(input_ids, wte, wpe):
    shape = input_ids.shape
    ids = input_ids.reshape(-1).astype(jnp.int32)
    out = _embed(ids, wte, wpe)
    return out.reshape(*shape, HIDDEN)


# speedup vs baseline: 2.5295x; 2.5295x over previous
import functools

import jax
import jax.numpy as jnp
from jax import lax
from jax.experimental import pallas as pl
from jax.experimental.pallas import tpu as pltpu
from jax.experimental.pallas import tpu_sc as plsc

HIDDEN = 1024
SEQ = 2048
NB = 4
NTOK = NB * SEQ
NC, NS = 2, 16
NW = NC * NS
PPW = SEQ // NW
C = 8
NPC = PPW // C
LANES = 16
VPR = HIDDEN // LANES

_mesh = plsc.VectorSubcoreMesh(core_axis_name="c", subcore_axis_name="s")


@functools.partial(
    pl.kernel,
    out_type=jax.ShapeDtypeStruct((NTOK, HIDDEN), jnp.float32),
    mesh=_mesh,
    scratch_types=[
        pltpu.VMEM((NB * PPW,), jnp.int32),
        pltpu.VMEM((2, NB, C, HIDDEN), jnp.float32),
        pltpu.VMEM((2, C, HIDDEN), jnp.float32),
        pltpu.SemaphoreType.DMA((2, NB)),
        pltpu.SemaphoreType.DMA((2, NB)),
        pltpu.SemaphoreType.DMA((2,)),
    ],
)
def _embed(ids_hbm, wte_hbm, wpe_hbm, out_hbm, idx_v, rows_v, wpe_v,
           gsem, osem, wsem):
    wid = lax.axis_index("s") * NC + lax.axis_index("c")
    pos0 = wid * PPW

    def wpe_copy(c, par):
        return pltpu.make_async_copy(
            wpe_hbm.at[pl.ds(pos0 + c * C, C)], wpe_v.at[par], wsem.at[par])

    def gather_copy(c, par, b):
        return pltpu.make_async_copy(
            wte_hbm.at[idx_v.at[pl.ds(b * PPW + c * C, C)]],
            rows_v.at[par, b], gsem.at[par, b])

    def store_copy(c, par, b):
        return pltpu.make_async_copy(
            rows_v.at[par, b],
            out_hbm.at[pl.ds(b * SEQ + pos0 + c * C, C)], osem.at[par, b])

    for b in range(NB):
        pltpu.sync_copy(ids_hbm.at[pl.ds(b * SEQ + pos0, PPW)],
                        idx_v.at[pl.ds(b * PPW, PPW)])
    wpe_copy(0, 0).start()
    for b in range(NB):
        gather_copy(0, 0, b).start()

    @pl.loop(0, NPC // 2)
    def _pair(cc):
        for par in (0, 1):
            c = 2 * cc + par
            nxt = 1 - par
            @pl.when(c < NPC - 1)
            def _():
                wpe_copy(c + 1, nxt).start()
            for b in range(NB):
                @pl.when(c < NPC - 1)
                def _():
                    @pl.when(c >= 1)
                    def _():
                        store_copy(c - 1, nxt, b).wait()
                    gather_copy(c + 1, nxt, b).start()
            wpe_copy(c, par).wait()
            for b in range(NB):
                gather_copy(c, par, b).wait()

                @pl.loop(0, C)
                def _row(r):
                    for j in range(VPR):
                        plsc.addupdate(
                            rows_v.at[par, b, r, pl.ds(j * LANES, LANES)],
                            wpe_v[par, r, pl.ds(j * LANES, LANES)])

                store_copy(c, par, b).start()

    for b in range(NB):
        store_copy(NPC - 1, (NPC - 1) % 2, b).wait()


def kernel(input_ids, wte, wpe):
    shape = input_ids.shape
    ids = input_ids.reshape(-1).astype(jnp.int32)
    out = _embed(ids, wte, wpe)
    return out.reshape(*shape, HIDDEN)


# speedup vs baseline: 2.5331x; 1.0014x over previous
import functools

import jax
import jax.numpy as jnp
from jax import lax
from jax.experimental import pallas as pl
from jax.experimental.pallas import tpu as pltpu
from jax.experimental.pallas import tpu_sc as plsc

HIDDEN = 1024
SEQ = 2048
NB = 4
NTOK = NB * SEQ
NC, NS = 2, 16
NW = NC * NS
PPW = SEQ // NW
C = 8
NPC = PPW // C
LANES = 16
VPR = HIDDEN // LANES

_mesh = plsc.VectorSubcoreMesh(core_axis_name="c", subcore_axis_name="s")


@functools.partial(
    pl.kernel,
    out_type=jax.ShapeDtypeStruct((NB, SEQ, HIDDEN), jnp.float32),
    mesh=_mesh,
    scratch_types=[
        pltpu.VMEM((NB * PPW,), jnp.int32),
        pltpu.VMEM((2, NB, C, HIDDEN), jnp.float32),
        pltpu.VMEM((2, C, HIDDEN), jnp.float32),
        pltpu.SemaphoreType.DMA((2, NB)),
        pltpu.SemaphoreType.DMA((2, NB)),
        pltpu.SemaphoreType.DMA((2,)),
    ],
)
def _embed(ids_hbm, wte_hbm, wpe_hbm, out_hbm, idx_v, rows_v, wpe_v,
           gsem, osem, wsem):
    wid = lax.axis_index("s") * NC + lax.axis_index("c")
    pos0 = wid * PPW

    def wpe_copy(c, par):
        return pltpu.make_async_copy(
            wpe_hbm.at[pl.ds(pos0 + c * C, C)], wpe_v.at[par], wsem.at[par])

    def gather_copy(c, par, b):
        return pltpu.make_async_copy(
            wte_hbm.at[idx_v.at[pl.ds(b * PPW + c * C, C)]],
            rows_v.at[par, b], gsem.at[par, b])

    def store_copy(c, par, b):
        return pltpu.make_async_copy(
            rows_v.at[par, b],
            out_hbm.at[b, pl.ds(pos0 + c * C, C)], osem.at[par, b])

    for b in range(NB):
        pltpu.sync_copy(ids_hbm.at[b, pl.ds(pos0, PPW)],
                        idx_v.at[pl.ds(b * PPW, PPW)])
    wpe_copy(0, 0).start()
    for b in range(NB):
        gather_copy(0, 0, b).start()

    @pl.loop(0, NPC // 2)
    def _pair(cc):
        for par in (0, 1):
            c = 2 * cc + par
            nxt = 1 - par
            @pl.when(c < NPC - 1)
            def _():
                wpe_copy(c + 1, nxt).start()
            for b in range(NB):
                @pl.when(c < NPC - 1)
                def _():
                    @pl.when(c >= 1)
                    def _():
                        store_copy(c - 1, nxt, b).wait()
                    gather_copy(c + 1, nxt, b).start()
            wpe_copy(c, par).wait()
            for b in range(NB):
                gather_copy(c, par, b).wait()

                @pl.loop(0, C)
                def _row(r):
                    for j in range(VPR):
                        plsc.addupdate(
                            rows_v.at[par, b, r, pl.ds(j * LANES, LANES)],
                            wpe_v[par, r, pl.ds(j * LANES, LANES)])

                store_copy(c, par, b).start()

    for b in range(NB):
        store_copy(NPC - 1, (NPC - 1) % 2, b).wait()


def kernel(input_ids, wte, wpe):
    return _embed(input_ids.astype(jnp.int32), wte, wpe)
